# Initial kernel scaffold; baseline (speedup 1.0000x reference)
import jax
import jax.numpy as jnp
from jax.experimental import pallas as pl


def kernel(x, atom_types, U3, U2, U1, W3, W2, W1):
    raise NotImplementedError("write your pallas kernel here")


# speedup vs baseline: 6.7958x; 6.7958x over previous
import jax
import jax.numpy as jnp
from jax.experimental import pallas as pl
from jax.experimental.pallas import tpu as pltpu

B = 512
C = 64
NLOUT = 16
L = 16
P3 = 4
P2 = 2
P1 = 1
QUAD = 4
NQ = B // QUAD
W = QUAD * C


def _sc_body(xq_ref, w3_ref, w2_ref, w1_ref, u3_ref, u2_ref, u1_ref, out_ref):
    xq = xq_ref[0]
    w3 = w3_ref[0]
    w2 = w2_ref[0]
    w1 = w1_ref[0]

    xx = jnp.concatenate([xq * xq[j:j + 1, :] for j in range(L)], axis=0)

    t = u1_ref[:, :] * w1

    for p in range(P3):
        rhs = xx * w3[p:p + 1, :]
        t = t + jnp.dot(u3_ref[p], rhs, preferred_element_type=jnp.float32)

    zw2 = jnp.concatenate([xq * w2[p:p + 1, :] for p in range(P2)], axis=0)
    t = t + jnp.dot(u2_ref[:, :], zw2, preferred_element_type=jnp.float32)

    t3 = t.reshape(NLOUT, L, W)
    out_ref[0] = jnp.sum(t3 * xq[None, :, :], axis=1)


def kernel(x, atom_types, U3, U2, U1, W3, W2, W1):
    W3g = jnp.take(W3, atom_types, axis=0)
    W2g = jnp.take(W2, atom_types, axis=0)
    W1g = jnp.take(W1, atom_types, axis=0)

    def quad_cols(a):
        n = a.shape[1]
        return a.reshape(NQ, QUAD, n, C).transpose(0, 2, 1, 3).reshape(NQ, n, W)

    xq = quad_cols(x)
    w3q = quad_cols(W3g)
    w2q = quad_cols(W2g)
    w1q = quad_cols(W1g)

    u3t = U3.transpose(4, 0, 1, 2, 3).reshape(P3, NLOUT * L, L * L)
    u2r = U2.transpose(0, 1, 3, 2).reshape(NLOUT * L, P2 * L)
    u1v = U1.reshape(NLOUT * L, P1)

    out = pl.pallas_call(
        _sc_body,
        grid=(NQ,),
        in_specs=[
            pl.BlockSpec((1, L, W), lambda q: (q, 0, 0)),
            pl.BlockSpec((1, P3, W), lambda q: (q, 0, 0)),
            pl.BlockSpec((1, P2, W), lambda q: (q, 0, 0)),
            pl.BlockSpec((1, P1, W), lambda q: (q, 0, 0)),
            pl.BlockSpec((P3, NLOUT * L, L * L), lambda q: (0, 0, 0)),
            pl.BlockSpec((NLOUT * L, P2 * L), lambda q: (0, 0)),
            pl.BlockSpec((NLOUT * L, P1), lambda q: (0, 0)),
        ],
        out_specs=pl.BlockSpec((1, NLOUT, W), lambda q: (q, 0, 0)),
        out_shape=jax.ShapeDtypeStruct((NQ, NLOUT, W), jnp.float32),
    )(xq, w3q, w2q, w1q, u3t, u2r, u1v)

    return out.reshape(NQ, NLOUT, QUAD, C).transpose(0, 2, 1, 3).reshape(B, NLOUT, C)
